# baseline (speedup 1.0000x reference)
import functools

import jax
import jax.numpy as jnp
from jax import lax
from jax.experimental import pallas as pl
from jax.experimental.pallas import tpu as pltpu
from jax.experimental.pallas import tpu_sc as plsc

N = 10000
D = 128
E = 320000
NR = 10240
NC, NS = 2, 16
NW = NC * NS
CHUNK = 96
NPART = 27
NPARTS = 4
NCHUNK = NPARTS * NPART
EPT = NCHUNK * CHUNK
EPAD = NW * EPT
ROWS_PER_TILE = NR // NS

_MESH = plsc.VectorSubcoreMesh(
    core_axis_name="c", subcore_axis_name="s", num_cores=NC, num_subcores=NS
)

f32 = jnp.float32


NSLOT = NR


@functools.partial(
    pl.kernel,
    out_type=jax.ShapeDtypeStruct((NW, NSLOT), f32),
    mesh=_MESH,
    scratch_types=[
        pltpu.VMEM((EPT,), jnp.int32),
        pltpu.VMEM((NSLOT,), f32),
    ],
    compiler_params=pltpu.CompilerParams(needs_layout_passes=False),
)
def _sc_degree(dst_hbm, out_hbm, dst_v, hist_v):
    c = lax.axis_index("c")
    s = lax.axis_index("s")
    wid = c * NS + s
    pltpu.sync_copy(dst_hbm.at[wid], dst_v)

    zeros16 = jnp.zeros((16,), f32)

    def zstep(i, carry):
        hist_v[pl.ds(i * 16, 16)] = zeros16
        return carry

    lax.fori_loop(0, NSLOT // 16, zstep, 0)

    ones = jnp.full((16,), 1.0, f32)

    def step(i, carry):
        v = dst_v[pl.ds(i * 16, 16)]
        plsc.addupdate_scatter(hist_v, [v], ones)
        return carry

    lax.fori_loop(0, EPT // 16, step, 0)
    pltpu.sync_copy(hist_v, out_hbm.at[wid])


@functools.partial(
    pl.kernel,
    out_type=jax.ShapeDtypeStruct((NC, NR, D), f32),
    mesh=_MESH,
    scratch_types=[
        pltpu.VMEM((NPART, CHUNK), jnp.int32),
        pltpu.VMEM((NPART, CHUNK), jnp.int32),
        pltpu.VMEM((CHUNK, D), f32),
        pltpu.VMEM((CHUNK, D), f32),
        pltpu.VMEM((CHUNK, D), f32),
        pltpu.VMEM_SHARED((NR, D), f32),
        pltpu.SemaphoreType.DMA,
        pltpu.SemaphoreType.DMA,
        pltpu.SemaphoreType.DMA,
    ],
)
def _sc_edge_pass(src_hbm, dst_hbm, g_hbm, out_hbm,
                  src_v, dst_v, rows0_v, rows1_v, rows2_v, acc_s,
                  sem0, sem1, sem2):
    c = lax.axis_index("c")
    s = lax.axis_index("s")
    wid = c * NS + s
    r0 = s * ROWS_PER_TILE
    @pl.when(c == 0)
    def _():
        pltpu.sync_copy(g_hbm.at[pl.ds(r0, ROWS_PER_TILE)],
                        acc_s.at[pl.ds(r0, ROWS_PER_TILE)])

    @pl.when(c != 0)
    def _():
        for k in range(ROWS_PER_TILE // 128):
            pltpu.sync_copy(g_hbm.at[pl.ds(N, 128)],
                            acc_s.at[pl.ds(r0 + k * 128, 128)])

    plsc.subcore_barrier()

    bufs = (rows0_v, rows1_v, rows2_v)
    sems = (sem0, sem1, sem2)

    def gather(j, b):
        pltpu.async_copy(g_hbm.at[src_v.at[j]], bufs[b], sems[b])

    def drain_scatter(j, b):
        pltpu.make_async_copy(g_hbm.at[src_v.at[j]], bufs[b], sems[b]).wait()
        pltpu.sync_copy(bufs[b], acc_s.at[dst_v.at[j]], add=True)

    for h in range(NPARTS):
        pltpu.sync_copy(src_hbm.at[wid, h], src_v)
        pltpu.sync_copy(dst_hbm.at[wid, h], dst_v)
        gather(0, 0)
        gather(1, 1)

        def step(i, carry):
            j = i * 3
            gather(j + 2, 2)
            drain_scatter(j, 0)

            @pl.when(i < NPART // 3 - 1)
            def _():
                gather(j + 3, 0)

            drain_scatter(j + 1, 1)

            @pl.when(i < NPART // 3 - 1)
            def _():
                gather(j + 4, 1)

            drain_scatter(j + 2, 2)
            return carry

        lax.fori_loop(0, NPART // 3, step, 0)

    plsc.subcore_barrier()
    pltpu.sync_copy(acc_s.at[pl.ds(r0, ROWS_PER_TILE)],
                    out_hbm.at[c, pl.ds(r0, ROWS_PER_TILE)])


_BS = 1024
_G = NR // _BS


def _row_spec():
    return pl.BlockSpec((_BS, D), lambda i: (i, 0))


def _col_spec():
    return pl.BlockSpec((_BS, 1), lambda i: (i, 0))


def _acc_spec(half):
    return pl.BlockSpec((1, _BS, D), lambda i, h=half: (h, i, 0))


def _tc_pre_body(degt_ref, emb_ref, w_ref, g_ref, dinv_ref):
    i = pl.program_id(0)
    deg = jnp.sum(degt_ref[...], axis=1, keepdims=True) + 1.0
    rid = lax.broadcasted_iota(jnp.int32, (_BS, 1), 0) + i * _BS
    dinv = jnp.where(rid < N, lax.rsqrt(deg), 0.0)
    h = jnp.dot(emb_ref[...], w_ref[...], preferred_element_type=f32)
    g_ref[...] = h * dinv
    dinv_ref[...] = dinv


def _tc_pre(degt, emb_pad, w1):
    return pl.pallas_call(
        _tc_pre_body,
        grid=(_G,),
        in_specs=[pl.BlockSpec((_BS, NW), lambda i: (i, 0)), _row_spec(),
                  pl.BlockSpec((D, D), lambda i: (0, 0))],
        out_specs=[_row_spec(), _col_spec()],
        out_shape=[jax.ShapeDtypeStruct((NR, D), f32),
                   jax.ShapeDtypeStruct((NR, 1), f32)],
    )(degt, emb_pad, w1)


def _tc_mid_body(a0_ref, a1_ref, dinv_ref, b_ref, w_ref, g_ref):
    dinv = dinv_ref[...]
    x = jnp.maximum((a0_ref[0] + a1_ref[0]) * dinv + b_ref[...], 0.0)
    g_ref[...] = jnp.dot(x, w_ref[...], preferred_element_type=f32) * dinv


def _tc_mid(acc, dinv, b1, w2):
    return pl.pallas_call(
        _tc_mid_body,
        grid=(_G,),
        in_specs=[_acc_spec(0), _acc_spec(1), _col_spec(),
                  pl.BlockSpec((1, D), lambda i: (0, 0)),
                  pl.BlockSpec((D, D), lambda i: (0, 0))],
        out_specs=_row_spec(),
        out_shape=jax.ShapeDtypeStruct((NR, D), f32),
    )(acc, acc, dinv, b1, w2)


def _tc_post_body(a0_ref, a1_ref, dinv_ref, b_ref, out_ref):
    out_ref[...] = (a0_ref[0] + a1_ref[0]) * dinv_ref[...] + b_ref[...]


def _tc_post(acc, dinv, b2):
    return pl.pallas_call(
        _tc_post_body,
        grid=(_G,),
        in_specs=[_acc_spec(0), _acc_spec(1), _col_spec(),
                  pl.BlockSpec((1, D), lambda i: (0, 0))],
        out_specs=_row_spec(),
        out_shape=jax.ShapeDtypeStruct((N, D), f32),
    )(acc, acc, dinv, b2)


def kernel(adj_t, emb, W1, b1, W2, b2):
    pad_i = jnp.arange(EPAD - E, dtype=jnp.int32)
    src = jnp.concatenate([adj_t[0].astype(jnp.int32), N + pad_i % (NR - N)])
    dst = jnp.concatenate([adj_t[1].astype(jnp.int32), N + pad_i % 128])
    src_t = src.reshape(NW, NPARTS, NPART, CHUNK)
    dst_t = dst.reshape(NW, NPARTS, NPART, CHUNK)
    dst_flat = dst.reshape(NW, EPT)

    emb_pad = jnp.zeros((NR, D), f32).at[:N].set(emb)

    degp = _sc_degree(dst_flat)
    degt = degp.T

    g1, dinv = _tc_pre(degt, emb_pad, W1)
    acc1 = _sc_edge_pass(src_t, dst_t, g1)
    g2 = _tc_mid(acc1, dinv, b1.reshape(1, D), W2)
    acc2 = _sc_edge_pass(src_t, dst_t, g2)
    return _tc_post(acc2, dinv, b2.reshape(1, D))


# speedup vs baseline: 1.0955x; 1.0955x over previous
import functools

import jax
import jax.numpy as jnp
from jax import lax
from jax.experimental import pallas as pl
from jax.experimental.pallas import tpu as pltpu
from jax.experimental.pallas import tpu_sc as plsc

N = 10000
D = 128
E = 320000
NR = 10240
NC, NS = 2, 16
NW = NC * NS
CHUNK = 80
NPART = 63
NPARTS = 2
NCHUNK = NPARTS * NPART
EPT = NCHUNK * CHUNK
EPAD = NW * EPT
ROWS_PER_TILE = NR // NS

_MESH = plsc.VectorSubcoreMesh(
    core_axis_name="c", subcore_axis_name="s", num_cores=NC, num_subcores=NS
)

f32 = jnp.float32


NSLOT = NR


@functools.partial(
    pl.kernel,
    out_type=jax.ShapeDtypeStruct((NW, NSLOT), f32),
    mesh=_MESH,
    scratch_types=[
        pltpu.VMEM((EPT,), jnp.int32),
        pltpu.VMEM((NSLOT,), f32),
    ],
    compiler_params=pltpu.CompilerParams(needs_layout_passes=False),
)
def _sc_degree(dst_hbm, out_hbm, dst_v, hist_v):
    c = lax.axis_index("c")
    s = lax.axis_index("s")
    wid = c * NS + s
    pltpu.sync_copy(dst_hbm.at[wid], dst_v)

    zeros16 = jnp.zeros((16,), f32)

    def zstep(i, carry):
        hist_v[pl.ds(i * 16, 16)] = zeros16
        return carry

    lax.fori_loop(0, NSLOT // 16, zstep, 0)

    ones = jnp.full((16,), 1.0, f32)

    def step(i, carry):
        v = dst_v[pl.ds(i * 16, 16)]
        plsc.addupdate_scatter(hist_v, [v], ones)
        return carry

    lax.fori_loop(0, EPT // 16, step, 0)
    pltpu.sync_copy(hist_v, out_hbm.at[wid])


@functools.partial(
    pl.kernel,
    out_type=jax.ShapeDtypeStruct((NC, NR, D), f32),
    mesh=_MESH,
    scratch_types=[
        pltpu.VMEM((NPART, CHUNK), jnp.int32),
        pltpu.VMEM((NPART, CHUNK), jnp.int32),
        pltpu.VMEM((CHUNK, D), f32),
        pltpu.VMEM((CHUNK, D), f32),
        pltpu.VMEM((CHUNK, D), f32),
        pltpu.VMEM_SHARED((NR, D), f32),
        pltpu.SemaphoreType.DMA,
        pltpu.SemaphoreType.DMA,
        pltpu.SemaphoreType.DMA,
    ],
)
def _sc_edge_pass(src_hbm, dst_hbm, g_hbm, out_hbm,
                  src_v, dst_v, rows0_v, rows1_v, rows2_v, acc_s,
                  sem0, sem1, sem2):
    c = lax.axis_index("c")
    s = lax.axis_index("s")
    wid = c * NS + s
    r0 = s * ROWS_PER_TILE
    @pl.when(c == 0)
    def _():
        pltpu.sync_copy(g_hbm.at[pl.ds(r0, ROWS_PER_TILE)],
                        acc_s.at[pl.ds(r0, ROWS_PER_TILE)])

    @pl.when(c != 0)
    def _():
        for k in range(ROWS_PER_TILE // 128):
            pltpu.sync_copy(g_hbm.at[pl.ds(N, 128)],
                            acc_s.at[pl.ds(r0 + k * 128, 128)])

    plsc.subcore_barrier()

    bufs = (rows0_v, rows1_v, rows2_v)
    sems = (sem0, sem1, sem2)

    def gather(j, b):
        pltpu.async_copy(g_hbm.at[src_v.at[j]], bufs[b], sems[b])

    def drain_scatter(j, b):
        pltpu.make_async_copy(g_hbm.at[src_v.at[j]], bufs[b], sems[b]).wait()
        pltpu.sync_copy(bufs[b], acc_s.at[dst_v.at[j]], add=True)

    for h in range(NPARTS):
        pltpu.sync_copy(src_hbm.at[wid, h], src_v)
        pltpu.sync_copy(dst_hbm.at[wid, h], dst_v)
        gather(0, 0)
        gather(1, 1)

        def step(i, carry):
            j = i * 3
            gather(j + 2, 2)
            drain_scatter(j, 0)

            @pl.when(i < NPART // 3 - 1)
            def _():
                gather(j + 3, 0)

            drain_scatter(j + 1, 1)

            @pl.when(i < NPART // 3 - 1)
            def _():
                gather(j + 4, 1)

            drain_scatter(j + 2, 2)
            return carry

        lax.fori_loop(0, NPART // 3, step, 0)

    plsc.subcore_barrier()
    pltpu.sync_copy(acc_s.at[pl.ds(r0, ROWS_PER_TILE)],
                    out_hbm.at[c, pl.ds(r0, ROWS_PER_TILE)])


_BS = 1024
_G = NR // _BS


def _row_spec():
    return pl.BlockSpec((_BS, D), lambda i: (i, 0))


def _col_spec():
    return pl.BlockSpec((_BS, 1), lambda i: (i, 0))


def _acc_spec(half):
    return pl.BlockSpec((1, _BS, D), lambda i, h=half: (h, i, 0))


def _tc_pre_body(degt_ref, emb_ref, w_ref, g_ref, dinv_ref):
    i = pl.program_id(0)
    deg = jnp.sum(degt_ref[...], axis=1, keepdims=True) + 1.0
    rid = lax.broadcasted_iota(jnp.int32, (_BS, 1), 0) + i * _BS
    dinv = jnp.where(rid < N, lax.rsqrt(deg), 0.0)
    h = jnp.dot(emb_ref[...], w_ref[...], preferred_element_type=f32)
    g_ref[...] = h * dinv
    dinv_ref[...] = dinv


def _tc_pre(degt, emb_pad, w1):
    return pl.pallas_call(
        _tc_pre_body,
        grid=(_G,),
        in_specs=[pl.BlockSpec((_BS, NW), lambda i: (i, 0)), _row_spec(),
                  pl.BlockSpec((D, D), lambda i: (0, 0))],
        out_specs=[_row_spec(), _col_spec()],
        out_shape=[jax.ShapeDtypeStruct((NR, D), f32),
                   jax.ShapeDtypeStruct((NR, 1), f32)],
    )(degt, emb_pad, w1)


def _tc_mid_body(a0_ref, a1_ref, dinv_ref, b_ref, w_ref, g_ref):
    dinv = dinv_ref[...]
    x = jnp.maximum((a0_ref[0] + a1_ref[0]) * dinv + b_ref[...], 0.0)
    g_ref[...] = jnp.dot(x, w_ref[...], preferred_element_type=f32) * dinv


def _tc_mid(acc, dinv, b1, w2):
    return pl.pallas_call(
        _tc_mid_body,
        grid=(_G,),
        in_specs=[_acc_spec(0), _acc_spec(1), _col_spec(),
                  pl.BlockSpec((1, D), lambda i: (0, 0)),
                  pl.BlockSpec((D, D), lambda i: (0, 0))],
        out_specs=_row_spec(),
        out_shape=jax.ShapeDtypeStruct((NR, D), f32),
    )(acc, acc, dinv, b1, w2)


def _tc_post_body(a0_ref, a1_ref, dinv_ref, b_ref, out_ref):
    out_ref[...] = (a0_ref[0] + a1_ref[0]) * dinv_ref[...] + b_ref[...]


def _tc_post(acc, dinv, b2):
    return pl.pallas_call(
        _tc_post_body,
        grid=(_G,),
        in_specs=[_acc_spec(0), _acc_spec(1), _col_spec(),
                  pl.BlockSpec((1, D), lambda i: (0, 0))],
        out_specs=_row_spec(),
        out_shape=jax.ShapeDtypeStruct((N, D), f32),
    )(acc, acc, dinv, b2)


def kernel(adj_t, emb, W1, b1, W2, b2):
    pad_i = jnp.arange(EPAD - E, dtype=jnp.int32)
    src = jnp.concatenate([adj_t[0].astype(jnp.int32), N + pad_i % (NR - N)])
    dst = jnp.concatenate([adj_t[1].astype(jnp.int32), N + pad_i % 128])
    src_t = src.reshape(NW, NPARTS, NPART, CHUNK)
    dst_t = dst.reshape(NW, NPARTS, NPART, CHUNK)
    dst_flat = dst.reshape(NW, EPT)

    emb_pad = jnp.zeros((NR, D), f32).at[:N].set(emb)

    degp = _sc_degree(dst_flat)
    degt = degp.T

    g1, dinv = _tc_pre(degt, emb_pad, W1)
    acc1 = _sc_edge_pass(src_t, dst_t, g1)
    g2 = _tc_mid(acc1, dinv, b1.reshape(1, D), W2)
    acc2 = _sc_edge_pass(src_t, dst_t, g2)
    return _tc_post(acc2, dinv, b2.reshape(1, D))
